# Initial kernel scaffold; baseline (speedup 1.0000x reference)
import functools

import jax
import jax.numpy as jnp
from jax import lax
from jax.experimental import pallas as pl
from jax.experimental.pallas import tpu as pltpu
from jax.experimental.pallas import tpu_sc as plsc

F32 = jnp.float32
I32 = jnp.int32

N0 = 10000
E = 320000
NF = 128
NG = 16
K1 = 5000
K2 = 2500
NDUM = 64
NW = 32
EPW = E // NW
C = 80


def _rup(v, m):
    return (v + m - 1) // m * m


def _block_rows(n):
    for br in (1024, 1000, 512, 256):
        if n % br == 0:
            return br
    return n


def _chunks(total, step):
    out = []
    off = 0
    while off < total:
        w = min(step, total - off)
        out.append((off, w))
        off += w
    return out


def _sc_mesh():
    return plsc.VectorSubcoreMesh(core_axis_name="c", subcore_axis_name="s")


def _make_deg(nr):
    nrt = nr // 16

    def body(ed_hbm, out_hbm, idxb, onesb, zedb, deg_sh):
        ci = lax.axis_index("c")
        si = lax.axis_index("s")
        wid = ci * 16 + si
        row1 = (lax.iota(I32, 16) == 0).astype(F32)
        zrow = jnp.zeros((16,), F32)
        for r in range(C):
            onesb[r, :] = row1
        for r in range(nrt):
            zedb[r, :] = zrow
        pltpu.sync_copy(zedb, deg_sh.at[pl.ds(si * nrt, nrt)])
        plsc.subcore_barrier()

        def step(i, carry):
            base = wid * EPW + i * C
            pltpu.sync_copy(ed_hbm.at[pl.ds(base, C)], idxb)
            pltpu.sync_copy(onesb, deg_sh.at[idxb], add=True)
            return carry

        lax.fori_loop(0, EPW // C, step, 0)
        plsc.subcore_barrier()
        pltpu.sync_copy(deg_sh.at[pl.ds(si * nrt, nrt)],
                        out_hbm.at[pl.ds(ci * nr + si * nrt, nrt)])

    return pl.kernel(
        body,
        out_type=jax.ShapeDtypeStruct((2 * nr, 16), F32),
        mesh=_sc_mesh(),
        scratch_types=[
            pltpu.VMEM((C,), I32),
            pltpu.VMEM((C, 16), F32),
            pltpu.VMEM((nrt, 16), F32),
            pltpu.VMEM_SHARED((nr, 16), F32),
        ],
    )


def _make_agg(nr):
    nrt = nr // 16

    def body(es_hbm, ed_hbm, hs_hbm, out_hbm, esb, edb, stage, agg_sh,
             sem1, sem2):
        ci = lax.axis_index("c")
        si = lax.axis_index("s")
        wid = ci * 16 + si
        zrow = jnp.zeros((16,), F32)
        for r in range(C):
            for q in range(NF // 16):
                stage[r, pl.ds(q * 16, 16)] = zrow
        for (off, w) in _chunks(nrt, C):
            pltpu.sync_copy(stage.at[pl.ds(0, w)],
                            agg_sh.at[pl.ds(si * nrt + off, w)])
        plsc.subcore_barrier()

        def step(i, carry):
            base = wid * EPW + i * C
            d1 = pltpu.async_copy(es_hbm.at[pl.ds(base, C)], esb, sem1)
            d2 = pltpu.async_copy(ed_hbm.at[pl.ds(base, C)], edb, sem2)
            d1.wait()
            pltpu.async_copy(hs_hbm.at[esb], stage, sem1).wait()
            d2.wait()
            pltpu.sync_copy(stage, agg_sh.at[edb], add=True)
            return carry

        lax.fori_loop(0, EPW // C, step, 0)
        plsc.subcore_barrier()
        pltpu.sync_copy(agg_sh.at[pl.ds(si * nrt, nrt)],
                        out_hbm.at[pl.ds(ci * nr + si * nrt, nrt)])

    return pl.kernel(
        body,
        out_type=jax.ShapeDtypeStruct((2 * nr, NF), F32),
        mesh=_sc_mesh(),
        scratch_types=[
            pltpu.VMEM((C,), I32),
            pltpu.VMEM((C,), I32),
            pltpu.VMEM((C, NF), F32),
            pltpu.VMEM_SHARED((nr, NF), F32),
            pltpu.SemaphoreType.DMA,
            pltpu.SemaphoreType.DMA,
        ],
    )


def _make_pool(nrm, kpad, nb, k_new):
    nrtm = nrm // 16
    kpw = kpad // NW

    def body(perm_s_hbm, perm_g_hbm, posk_hbm, hr_hbm, batch_hbm,
             es_hbm, ed_hbm,
             xp_hbm, b2_hbm, esn_hbm, edn_hbm,
             map_sh, map_v, batch_v, pv, posv, stage, b2v,
             esv, edv, esnv, ednv, negv, sem1, sem2):
        ci = lax.axis_index("c")
        si = lax.axis_index("s")
        wid = ci * 16 + si
        neg = jnp.full((16,), -1, I32)
        for r in range(nrtm // 16):
            negv[pl.ds(r * 16, 16)] = neg
        pltpu.sync_copy(negv, map_sh.at[pl.ds(wid * (nrm // NW),
                                              nrm // NW)])
        plsc.subcore_barrier()
        for j in range(kpw // C):
            base = wid * kpw + j * C
            pltpu.sync_copy(perm_s_hbm.at[pl.ds(base, C)], pv)
            pltpu.sync_copy(posk_hbm.at[pl.ds(base, C)], posv)
            pltpu.sync_copy(posv, map_sh.at[pv])
        plsc.subcore_barrier()
        pltpu.sync_copy(map_sh, map_v)
        pltpu.sync_copy(batch_hbm, batch_v.at[pl.ds(0, nb)])
        for j in range(kpw // C):
            base = wid * kpw + j * C
            pltpu.sync_copy(perm_g_hbm.at[pl.ds(base, C)], pv)
            pltpu.async_copy(hr_hbm.at[pv], stage, sem1).wait()
            pltpu.sync_copy(stage, xp_hbm.at[pl.ds(base, C)])
            for q in range(C // 16):
                idx16 = pv[pl.ds(q * 16, 16)]
                b2v[pl.ds(q * 16, 16)] = plsc.load_gather(batch_v, [idx16])
            pltpu.sync_copy(b2v, b2_hbm.at[pl.ds(base, C)])

        def step(i, carry):
            base = wid * EPW + i * C
            d1 = pltpu.async_copy(es_hbm.at[pl.ds(base, C)], esv, sem1)
            d2 = pltpu.async_copy(ed_hbm.at[pl.ds(base, C)], edv, sem2)
            d1.wait()
            d2.wait()
            for q in range(C // 16):
                s16 = esv[pl.ds(q * 16, 16)]
                d16 = edv[pl.ds(q * 16, 16)]
                ms = plsc.load_gather(map_v, [s16])
                md = plsc.load_gather(map_v, [d16])
                ok = (ms >= 0) & (md >= 0)
                sp = (lax.iota(I32, 16) + (base + q * 16)) & (NDUM - 1)
                esnv[pl.ds(q * 16, 16)] = jnp.where(ok, ms, sp)
                ednv[pl.ds(q * 16, 16)] = jnp.where(ok, md, k_new + sp)
            pltpu.sync_copy(esnv, esn_hbm.at[pl.ds(base, C)])
            pltpu.sync_copy(ednv, edn_hbm.at[pl.ds(base, C)])
            return carry

        lax.fori_loop(0, EPW // C, step, 0)

    nbp = _rup(nb, 8)
    return pl.kernel(
        body,
        out_type=(
            jax.ShapeDtypeStruct((kpad, NF), F32),
            jax.ShapeDtypeStruct((kpad,), I32),
            jax.ShapeDtypeStruct((E,), I32),
            jax.ShapeDtypeStruct((E,), I32),
        ),
        mesh=_sc_mesh(),
        scratch_types=[
            pltpu.VMEM_SHARED((nrm,), I32),
            pltpu.VMEM((nrm,), I32),
            pltpu.VMEM((nbp,), I32),
            pltpu.VMEM((C,), I32),
            pltpu.VMEM((C,), I32),
            pltpu.VMEM((C, NF), F32),
            pltpu.VMEM((C,), I32),
            pltpu.VMEM((C,), I32),
            pltpu.VMEM((C,), I32),
            pltpu.VMEM((C,), I32),
            pltpu.VMEM((C,), I32),
            pltpu.VMEM((nrtm,), I32),
            pltpu.SemaphoreType.DMA,
            pltpu.SemaphoreType.DMA,
        ],
    )


def _mm_body(h_ref, w_ref, b_ref, o_ref):
    o_ref[...] = jax.lax.dot(
        h_ref[...], w_ref[...], precision=lax.Precision.HIGHEST,
        preferred_element_type=F32) + b_ref[...]


def _tc_mm(h, w, b):
    n = h.shape[0]
    br = _block_rows(n)
    return pl.pallas_call(
        _mm_body,
        grid=(n // br,),
        in_specs=[
            pl.BlockSpec((br, NF), lambda i: (i, 0)),
            pl.BlockSpec((NF, NF), lambda i: (0, 0)),
            pl.BlockSpec((1, NF), lambda i: (0, 0)),
        ],
        out_specs=pl.BlockSpec((br, NF), lambda i: (i, 0)),
        out_shape=jax.ShapeDtypeStruct((n, NF), F32),
    )(h, w, b.reshape(1, NF))


def _scale_body(hh_ref, d0_ref, d1_ref, hs_ref, rn_ref, id_ref):
    deg = d0_ref[...] + d1_ref[...] + 1.0
    rn = lax.rsqrt(deg)
    rn_ref[...] = rn
    id_ref[...] = 1.0 / deg
    hs_ref[...] = hh_ref[...] * rn


def _tc_scale(hh, d0, d1):
    n = hh.shape[0]
    br = _block_rows(n)
    return pl.pallas_call(
        _scale_body,
        grid=(n // br,),
        in_specs=[
            pl.BlockSpec((br, NF), lambda i: (i, 0)),
            pl.BlockSpec((br, 1), lambda i: (i, 0)),
            pl.BlockSpec((br, 1), lambda i: (i, 0)),
        ],
        out_specs=[
            pl.BlockSpec((br, NF), lambda i: (i, 0)),
            pl.BlockSpec((br, 1), lambda i: (i, 0)),
            pl.BlockSpec((br, 1), lambda i: (i, 0)),
        ],
        out_shape=[
            jax.ShapeDtypeStruct((n, NF), F32),
            jax.ShapeDtypeStruct((n, 1), F32),
            jax.ShapeDtypeStruct((n, 1), F32),
        ],
    )(hh, d0, d1)


def _post_body(p0_ref, p1_ref, hh_ref, rn_ref, id_ref, pv_ref,
               sc_ref, hr_ref):
    agg = (p0_ref[...] + p1_ref[...]) * rn_ref[...] + hh_ref[...] * id_ref[...]
    sc_ref[...] = jnp.tanh(jax.lax.dot(
        agg, pv_ref[...], precision=lax.Precision.HIGHEST,
        preferred_element_type=F32))
    hr_ref[...] = jnp.maximum(agg, 0.0)


def _tc_post(pp0, pp1, hh, rn, ideg, pvec):
    n = hh.shape[0]
    br = _block_rows(n)
    return pl.pallas_call(
        _post_body,
        grid=(n // br,),
        in_specs=[
            pl.BlockSpec((br, NF), lambda i: (i, 0)),
            pl.BlockSpec((br, NF), lambda i: (i, 0)),
            pl.BlockSpec((br, NF), lambda i: (i, 0)),
            pl.BlockSpec((br, 1), lambda i: (i, 0)),
            pl.BlockSpec((br, 1), lambda i: (i, 0)),
            pl.BlockSpec((NF, 1), lambda i: (0, 0)),
        ],
        out_specs=[
            pl.BlockSpec((br, 1), lambda i: (i, 0)),
            pl.BlockSpec((br, NF), lambda i: (i, 0)),
        ],
        out_shape=[
            jax.ShapeDtypeStruct((n, 1), F32),
            jax.ShapeDtypeStruct((n, NF), F32),
        ],
    )(pp0, pp1, hh, rn, ideg, pvec.reshape(NF, 1))


def _make_read_body(k):
    def body(xp_ref, v_ref, b2_ref, xn_ref, x_ref):
        xn = xp_ref[...] * jnp.tanh(v_ref[...])
        xn_ref[...] = xn
        rows = lax.broadcasted_iota(I32, b2_ref.shape, 0)
        valid = rows < k
        for g in range(NG):
            m = valid & (b2_ref[...] == g)
            mf = m.astype(F32)
            cnt = jnp.sum(mf)
            sm = jnp.sum(xn * mf, axis=0, keepdims=True)
            mx = jnp.max(jnp.where(m, xn, -jnp.inf), axis=0, keepdims=True)
            mx = jnp.where(cnt > 0, mx, 0.0)
            x_ref[pl.ds(g, 1), pl.ds(0, NF)] = mx
            x_ref[pl.ds(g, 1), pl.ds(NF, NF)] = sm / jnp.maximum(cnt, 1.0)
    return body


def _tc_read(xp, vals_col, b2_col, k):
    n = xp.shape[0]
    return pl.pallas_call(
        _make_read_body(k),
        out_shape=[
            jax.ShapeDtypeStruct((n, NF), F32),
            jax.ShapeDtypeStruct((NG, 2 * NF), F32),
        ],
    )(xp, vals_col, b2_col)


def _final_body(x1_ref, x2_ref, x3_ref, w1_ref, b1_ref, w2_ref, b2_ref,
                w3_ref, b3_ref, o_ref):
    z = (jnp.maximum(x1_ref[...], 0.0) + jnp.maximum(x2_ref[...], 0.0)
         + jnp.maximum(x3_ref[...], 0.0))
    hp = lax.Precision.HIGHEST
    z = jnp.maximum(jax.lax.dot(z, w1_ref[...], precision=hp,
                                preferred_element_type=F32) + b1_ref[...], 0.0)
    z = jnp.maximum(jax.lax.dot(z, w2_ref[...], precision=hp,
                                preferred_element_type=F32) + b2_ref[...], 0.0)
    z = jax.lax.dot(z, w3_ref[...], precision=hp,
                    preferred_element_type=F32) + b3_ref[...]
    m = jnp.max(z, axis=-1, keepdims=True)
    lse = jnp.log(jnp.sum(jnp.exp(z - m), axis=-1, keepdims=True)) + m
    o_ref[...] = z - lse


def _tc_final(x1, x2, x3, lW1, lb1, lW2p, lb2p, lW3p, lb3p):
    return pl.pallas_call(
        _final_body,
        out_shape=jax.ShapeDtypeStruct((NG, NF), F32),
    )(x1, x2, x3, lW1, lb1.reshape(1, NF), lW2p, lb2p, lW3p, lb3p)


def _layer(h, es, ed, W, b, pvec, nr):
    degp = _make_deg(nr)(ed)
    hh = _tc_mm(h, W, b)
    n = h.shape[0]
    d0 = lax.slice(degp, (0, 0), (n, 1))
    d1 = lax.slice(degp, (nr, 0), (nr + n, 1))
    hs, rn, ideg = _tc_scale(hh, d0, d1)
    aggp = _make_agg(nr)(es, ed, hs)
    score, hr = _tc_post(aggp[:n], aggp[nr:nr + n], hh, rn, ideg, pvec)
    return score, hr


def _pool(score, hr, batch_old, es, ed, n_old, nrm, k):
    kpad = _rup(k, 256)
    pad = kpad - k
    vals, perm = lax.top_k(score[:n_old, 0], k)
    spread = jnp.arange(pad, dtype=I32) % NDUM
    perm_s = jnp.concatenate([perm, n_old + spread])
    perm_g = jnp.concatenate([perm, spread])
    posk = jnp.concatenate([jnp.arange(k, dtype=I32),
                            jnp.full((pad,), -1, I32)])
    nb = batch_old.shape[0]
    xp, b2, esn, edn = _make_pool(nrm, kpad, nb, k)(
        perm_s, perm_g, posk, hr, batch_old, es, ed)
    vals_col = jnp.concatenate([vals, jnp.zeros((pad,), F32)]).reshape(kpad, 1)
    return xp, vals_col, b2, esn, edn


def kernel(x, edge_index, batch, W1, b1, p1, W2, b2, p2, W3, b3, p3,
           lW1, lb1, lW2, lb2, lW3, lb3):
    src = edge_index[0].astype(I32)
    dst = edge_index[1].astype(I32)
    batch = batch.astype(I32)

    nr1 = _rup(N0 + NDUM, 128)
    nr2 = _rup(max(K1 + NDUM, _rup(K1, 256)), 128)
    nr3 = _rup(max(K2 + NDUM, _rup(K2, 256)), 128)

    score1, hr1 = _layer(x, src, dst, W1, b1, p1, nr1)
    xp1, v1c, b21, es2, ed2 = _pool(score1, hr1, batch, src, dst,
                                    N0, nr1, K1)
    xn1, x1 = _tc_read(xp1, v1c, b21.reshape(-1, 1), K1)

    score2, hr2 = _layer(xn1, es2, ed2, W2, b2, p2, nr2)
    xp2, v2c, b22, es3, ed3 = _pool(score2, hr2, b21, es2, ed2,
                                    K1, nr2, K2)
    xn2, x2 = _tc_read(xp2, v2c, b22.reshape(-1, 1), K2)

    _, hr3 = _layer(xn2, es3, ed3, W3, b3, p3, nr3)
    big = jnp.full((xn2.shape[0], 1), 20.0, F32)
    _, x3 = _tc_read(hr3, big, b22.reshape(-1, 1), K2)

    lW2p = jnp.pad(lW2, ((0, 0), (0, NF - lW2.shape[1])))
    lb2p = jnp.pad(lb2, (0, NF - lb2.shape[0])).reshape(1, NF)
    lW3p = jnp.pad(lW3, ((0, NF - lW3.shape[0]), (0, NF - lW3.shape[1])))
    lb3p = jnp.concatenate(
        [lb3, jnp.full((NF - lb3.shape[0],), -1e30, F32)]).reshape(1, NF)
    out = _tc_final(x1, x2, x3, lW1, lb1, lW2p, lb2p, lW3p, lb3p)
    return out[:, :lb3.shape[0]]


# speedup vs baseline: 16.8699x; 16.8699x over previous
import jax
import jax.numpy as jnp
from jax import lax
from jax.experimental import pallas as pl
from jax.experimental.pallas import tpu as pltpu
from jax.experimental.pallas import tpu_sc as plsc

F32 = jnp.float32
I32 = jnp.int32

N0 = 10000
E = 320000
NF = 128
NG = 16
K1 = 5000
K2 = 2500
NDUM = 64
NW = 32
EPW = E // NW
C = 80


def _rup(v, m):
    return (v + m - 1) // m * m


def _block_rows(n):
    for br in (1024, 1000, 512, 256):
        if n % br == 0:
            return br
    return n


def _chunks(total, step):
    out = []
    off = 0
    while off < total:
        w = min(step, total - off)
        out.append((off, w))
        off += w
    return out


_SC_PARAMS = pltpu.CompilerParams(needs_layout_passes=False)


def _z16f():
    return jnp.zeros((16,), F32)


def _z16i():
    return jnp.zeros((16,), I32)


def _sc_mesh():
    return plsc.VectorSubcoreMesh(core_axis_name="c", subcore_axis_name="s")


def _fill2d(buf, rows, val):
    for r in range(rows):
        for q in range(NF // 16):
            buf[r, pl.ds(q * 16, 16)] = val


def _make_deg(nr):
    nrt = nr // 16

    def body(ed_hbm, out_hbm, idxb, onesb, zbuf, deg_sh):
        ci = lax.axis_index("c")
        si = lax.axis_index("s")
        wid = ci * 16 + si
        _fill2d(zbuf, C, _z16f())
        _fill2d(onesb, C, _z16f())
        one16 = jnp.ones((16,), F32)
        for q in range(C // 16):
            plsc.store_scatter(onesb, [lax.iota(I32, 16) + q * 16, _z16i()],
                               one16)
        for (off, w) in _chunks(nrt, C):
            pltpu.sync_copy(zbuf.at[pl.ds(0, w)],
                            deg_sh.at[pl.ds(si * nrt + off, w)])
        plsc.subcore_barrier()

        def step(i, carry):
            base = wid * EPW + i * C
            pltpu.sync_copy(ed_hbm.at[pl.ds(base, C)], idxb)
            pltpu.sync_copy(onesb, deg_sh.at[idxb], add=True)
            return carry

        lax.fori_loop(0, EPW // C, step, 0)
        plsc.subcore_barrier()
        pltpu.sync_copy(deg_sh.at[pl.ds(si * nrt, nrt)],
                        out_hbm.at[pl.ds(ci * nr + si * nrt, nrt)])

    return pl.kernel(
        body,
        out_type=jax.ShapeDtypeStruct((2 * nr, NF), F32),
        mesh=_sc_mesh(),
        compiler_params=_SC_PARAMS,
        scratch_types=[
            pltpu.VMEM((C,), I32),
            pltpu.VMEM((C, NF), F32),
            pltpu.VMEM((C, NF), F32),
            pltpu.VMEM_SHARED((nr, NF), F32),
        ],
    )


def _make_agg(nr):
    nrt = nr // 16

    def body(es_hbm, ed_hbm, hs_hbm, out_hbm, esb, edb, stage, agg_sh,
             sem1, sem2):
        ci = lax.axis_index("c")
        si = lax.axis_index("s")
        wid = ci * 16 + si
        _fill2d(stage, C, _z16f())
        for (off, w) in _chunks(nrt, C):
            pltpu.sync_copy(stage.at[pl.ds(0, w)],
                            agg_sh.at[pl.ds(si * nrt + off, w)])
        plsc.subcore_barrier()

        def step(i, carry):
            base = wid * EPW + i * C
            d1 = pltpu.async_copy(es_hbm.at[pl.ds(base, C)], esb, sem1)
            d2 = pltpu.async_copy(ed_hbm.at[pl.ds(base, C)], edb, sem2)
            d1.wait()
            pltpu.async_copy(hs_hbm.at[esb], stage, sem1).wait()
            d2.wait()
            pltpu.sync_copy(stage, agg_sh.at[edb], add=True)
            return carry

        lax.fori_loop(0, EPW // C, step, 0)
        plsc.subcore_barrier()
        pltpu.sync_copy(agg_sh.at[pl.ds(si * nrt, nrt)],
                        out_hbm.at[pl.ds(ci * nr + si * nrt, nrt)])

    return pl.kernel(
        body,
        out_type=jax.ShapeDtypeStruct((2 * nr, NF), F32),
        mesh=_sc_mesh(),
        compiler_params=_SC_PARAMS,
        scratch_types=[
            pltpu.VMEM((C,), I32),
            pltpu.VMEM((C,), I32),
            pltpu.VMEM((C, NF), F32),
            pltpu.VMEM_SHARED((nr, NF), F32),
            pltpu.SemaphoreType.DMA,
            pltpu.SemaphoreType.DMA,
        ],
    )


def _make_pool(nrm, kpad, nb, k_new):
    nrtm = nrm // 16
    kpw = kpad // 16

    def body(perm_s_hbm, perm_g_hbm, posk_hbm, hr_hbm, batch_hbm,
             es_hbm, ed_hbm,
             xp_hbm, b2_hbm, esn_hbm, edn_hbm,
             map2_sh, mapf_sh, map_v, batch_v, pv, posv, stage, buf2,
             b2v, esv, edv, esnv, ednv, flatb, sem1, sem2):
        ci = lax.axis_index("c")
        si = lax.axis_index("s")
        wid = ci * 16 + si
        neg16 = jnp.full((16,), -1, I32)
        _fill2d(buf2, C, neg16)
        for (off, w) in _chunks(nrtm, C):
            pltpu.sync_copy(buf2.at[pl.ds(0, w)],
                            map2_sh.at[pl.ds(si * nrtm + off, w)])
        plsc.subcore_barrier()
        for j in range(kpw // C):
            base = si * kpw + j * C
            pltpu.sync_copy(perm_s_hbm.at[pl.ds(base, C)], pv)
            pltpu.sync_copy(posk_hbm.at[pl.ds(base, C)], posv)
            for q in range(C // 16):
                plsc.store_scatter(
                    buf2, [lax.iota(I32, 16) + q * 16, _z16i()],
                    posv[pl.ds(q * 16, 16)])
            pltpu.sync_copy(buf2, map2_sh.at[pv])
        plsc.subcore_barrier()
        for (off, w) in _chunks(nrtm, C):
            pltpu.sync_copy(map2_sh.at[pl.ds(si * nrtm + off, w)],
                            buf2.at[pl.ds(0, w)])
            for g in range(C // 16):
                if g * 16 < w:
                    flatb[pl.ds(g * 16, 16)] = plsc.load_gather(
                        buf2, [lax.iota(I32, 16) + g * 16, _z16i()])
            pltpu.sync_copy(flatb.at[pl.ds(0, w)],
                            mapf_sh.at[pl.ds(si * nrtm + off, w)])
        plsc.subcore_barrier()
        pltpu.sync_copy(mapf_sh, map_v)
        pltpu.sync_copy(batch_hbm, batch_v)
        for j in range((kpad // NW) // C):
            base = wid * (kpad // NW) + j * C
            pltpu.sync_copy(perm_g_hbm.at[pl.ds(base, C)], pv)
            pltpu.async_copy(hr_hbm.at[pv], stage, sem1).wait()
            pltpu.sync_copy(stage, xp_hbm.at[pl.ds(base, C)])
            for q in range(C // 16):
                idx16 = pv[pl.ds(q * 16, 16)]
                b2v[pl.ds(q * 16, 16)] = plsc.load_gather(batch_v, [idx16])
            pltpu.sync_copy(b2v, b2_hbm.at[pl.ds(base, C)])

        def step(i, carry):
            base = wid * EPW + i * C
            d1 = pltpu.async_copy(es_hbm.at[pl.ds(base, C)], esv, sem1)
            d2 = pltpu.async_copy(ed_hbm.at[pl.ds(base, C)], edv, sem2)
            d1.wait()
            d2.wait()
            for q in range(C // 16):
                s16 = esv[pl.ds(q * 16, 16)]
                d16 = edv[pl.ds(q * 16, 16)]
                ms = plsc.load_gather(map_v, [s16])
                md = plsc.load_gather(map_v, [d16])
                ok = (ms >= 0) & (md >= 0)
                sp = (lax.iota(I32, 16) + (base + q * 16)) & (NDUM - 1)
                esnv[pl.ds(q * 16, 16)] = jnp.where(ok, ms, sp)
                ednv[pl.ds(q * 16, 16)] = jnp.where(ok, md, k_new + sp)
            pltpu.sync_copy(esnv, esn_hbm.at[pl.ds(base, C)])
            pltpu.sync_copy(ednv, edn_hbm.at[pl.ds(base, C)])
            return carry

        lax.fori_loop(0, EPW // C, step, 0)

    return pl.kernel(
        body,
        out_type=(
            jax.ShapeDtypeStruct((kpad, NF), F32),
            jax.ShapeDtypeStruct((kpad,), I32),
            jax.ShapeDtypeStruct((E,), I32),
            jax.ShapeDtypeStruct((E,), I32),
        ),
        mesh=_sc_mesh(),
        compiler_params=_SC_PARAMS,
        scratch_types=[
            pltpu.VMEM_SHARED((nrm, NF), I32),
            pltpu.VMEM_SHARED((nrm,), I32),
            pltpu.VMEM((nrm,), I32),
            pltpu.VMEM((nb,), I32),
            pltpu.VMEM((C,), I32),
            pltpu.VMEM((C,), I32),
            pltpu.VMEM((C, NF), F32),
            pltpu.VMEM((C, NF), I32),
            pltpu.VMEM((C,), I32),
            pltpu.VMEM((C,), I32),
            pltpu.VMEM((C,), I32),
            pltpu.VMEM((C,), I32),
            pltpu.VMEM((C,), I32),
            pltpu.VMEM((C,), I32),
            pltpu.SemaphoreType.DMA,
            pltpu.SemaphoreType.DMA,
        ],
    )


def _mm_body(h_ref, w_ref, b_ref, o_ref):
    o_ref[...] = jax.lax.dot(
        h_ref[...], w_ref[...], precision=lax.Precision.HIGHEST,
        preferred_element_type=F32) + b_ref[...]


def _tc_mm(h, w, b):
    n = h.shape[0]
    br = _block_rows(n)
    return pl.pallas_call(
        _mm_body,
        grid=(n // br,),
        in_specs=[
            pl.BlockSpec((br, NF), lambda i: (i, 0)),
            pl.BlockSpec((NF, NF), lambda i: (0, 0)),
            pl.BlockSpec((1, NF), lambda i: (0, 0)),
        ],
        out_specs=pl.BlockSpec((br, NF), lambda i: (i, 0)),
        out_shape=jax.ShapeDtypeStruct((n, NF), F32),
    )(h, w, b.reshape(1, NF))


def _scale_body(hh_ref, d0_ref, d1_ref, hs_ref, rn_ref, id_ref):
    deg = d0_ref[...] + d1_ref[...] + 1.0
    rn = lax.rsqrt(deg)
    rn_ref[...] = rn
    id_ref[...] = 1.0 / deg
    hs_ref[...] = hh_ref[...] * rn


def _tc_scale(hh, d0, d1):
    n = hh.shape[0]
    br = _block_rows(n)
    return pl.pallas_call(
        _scale_body,
        grid=(n // br,),
        in_specs=[
            pl.BlockSpec((br, NF), lambda i: (i, 0)),
            pl.BlockSpec((br, 1), lambda i: (i, 0)),
            pl.BlockSpec((br, 1), lambda i: (i, 0)),
        ],
        out_specs=[
            pl.BlockSpec((br, NF), lambda i: (i, 0)),
            pl.BlockSpec((br, 1), lambda i: (i, 0)),
            pl.BlockSpec((br, 1), lambda i: (i, 0)),
        ],
        out_shape=[
            jax.ShapeDtypeStruct((n, NF), F32),
            jax.ShapeDtypeStruct((n, 1), F32),
            jax.ShapeDtypeStruct((n, 1), F32),
        ],
    )(hh, d0, d1)


def _post_body(p0_ref, p1_ref, hh_ref, rn_ref, id_ref, pv_ref,
               sc_ref, hr_ref):
    agg = (p0_ref[...] + p1_ref[...]) * rn_ref[...] + hh_ref[...] * id_ref[...]
    sc_ref[...] = jnp.tanh(jax.lax.dot(
        agg, pv_ref[...], precision=lax.Precision.HIGHEST,
        preferred_element_type=F32))
    hr_ref[...] = jnp.maximum(agg, 0.0)


def _tc_post(pp0, pp1, hh, rn, ideg, pvec):
    n = hh.shape[0]
    br = _block_rows(n)
    return pl.pallas_call(
        _post_body,
        grid=(n // br,),
        in_specs=[
            pl.BlockSpec((br, NF), lambda i: (i, 0)),
            pl.BlockSpec((br, NF), lambda i: (i, 0)),
            pl.BlockSpec((br, NF), lambda i: (i, 0)),
            pl.BlockSpec((br, 1), lambda i: (i, 0)),
            pl.BlockSpec((br, 1), lambda i: (i, 0)),
            pl.BlockSpec((NF, 1), lambda i: (0, 0)),
        ],
        out_specs=[
            pl.BlockSpec((br, 1), lambda i: (i, 0)),
            pl.BlockSpec((br, NF), lambda i: (i, 0)),
        ],
        out_shape=[
            jax.ShapeDtypeStruct((n, 1), F32),
            jax.ShapeDtypeStruct((n, NF), F32),
        ],
    )(pp0, pp1, hh, rn, ideg, pvec.reshape(NF, 1))


def _make_read_body(k, br, nb):
    def body(xp_ref, v_ref, b2r_ref, b2c_ref, xn_ref, x_ref, acc, cnt):
        i = pl.program_id(0)

        @pl.when(i == 0)
        def _init():
            acc[:, pl.ds(0, NF)] = jnp.full((NG, NF), -1e30, F32)
            acc[:, pl.ds(NF, NF)] = jnp.zeros((NG, NF), F32)
            cnt[...] = jnp.zeros((NG, 1), F32)

        xn = xp_ref[...] * jnp.tanh(v_ref[...])
        xn_ref[...] = xn
        rows_r = lax.broadcasted_iota(I32, (1, br), 1) + i * br
        valid_r = rows_r < k
        gids = lax.broadcasted_iota(I32, (NG, br), 0)
        onehot = ((b2r_ref[...] == gids) & valid_r).astype(F32)
        acc[:, pl.ds(NF, NF)] += jax.lax.dot(
            onehot, xn, precision=lax.Precision.HIGHEST,
            preferred_element_type=F32)
        cnt[...] += jnp.sum(onehot, axis=1, keepdims=True)
        rows_c = lax.broadcasted_iota(I32, (br, 1), 0) + i * br
        valid_c = rows_c < k
        for g in range(NG):
            m = valid_c & (b2c_ref[...] == g)
            mx = jnp.max(jnp.where(m, xn, -1e30), axis=0, keepdims=True)
            acc[pl.ds(g, 1), pl.ds(0, NF)] = jnp.maximum(
                acc[pl.ds(g, 1), pl.ds(0, NF)], mx)

        @pl.when(i == nb - 1)
        def _fin():
            c = cnt[...]
            mx = jnp.where(c > 0, acc[:, pl.ds(0, NF)], 0.0)
            x_ref[:, pl.ds(0, NF)] = mx
            x_ref[:, pl.ds(NF, NF)] = (acc[:, pl.ds(NF, NF)]
                                       / jnp.maximum(c, 1.0))
    return body


def _tc_read(xp, vals_col, b2, k):
    n = xp.shape[0]
    br = 512
    nb = n // br
    return pl.pallas_call(
        _make_read_body(k, br, nb),
        grid=(nb,),
        in_specs=[
            pl.BlockSpec((br, NF), lambda i: (i, 0)),
            pl.BlockSpec((br, 1), lambda i: (i, 0)),
            pl.BlockSpec((1, br), lambda i: (0, i)),
            pl.BlockSpec((br, 1), lambda i: (i, 0)),
        ],
        out_specs=[
            pl.BlockSpec((br, NF), lambda i: (i, 0)),
            pl.BlockSpec((NG, 2 * NF), lambda i: (0, 0)),
        ],
        out_shape=[
            jax.ShapeDtypeStruct((n, NF), F32),
            jax.ShapeDtypeStruct((NG, 2 * NF), F32),
        ],
        scratch_shapes=[
            pltpu.VMEM((NG, 2 * NF), F32),
            pltpu.VMEM((NG, 1), F32),
        ],
    )(xp, vals_col, b2.reshape(1, n), b2.reshape(n, 1))


def _final_body(x1_ref, x2_ref, x3_ref, w1_ref, b1_ref, w2_ref, b2_ref,
                w3_ref, b3_ref, o_ref):
    z = (jnp.maximum(x1_ref[...], 0.0) + jnp.maximum(x2_ref[...], 0.0)
         + jnp.maximum(x3_ref[...], 0.0))
    hp = lax.Precision.HIGHEST
    z = jnp.maximum(jax.lax.dot(z, w1_ref[...], precision=hp,
                                preferred_element_type=F32) + b1_ref[...], 0.0)
    z = jnp.maximum(jax.lax.dot(z, w2_ref[...], precision=hp,
                                preferred_element_type=F32) + b2_ref[...], 0.0)
    z = jax.lax.dot(z, w3_ref[...], precision=hp,
                    preferred_element_type=F32) + b3_ref[...]
    m = jnp.max(z, axis=-1, keepdims=True)
    lse = jnp.log(jnp.sum(jnp.exp(z - m), axis=-1, keepdims=True)) + m
    o_ref[...] = z - lse


def _tc_final(x1, x2, x3, lW1, lb1, lW2p, lb2p, lW3p, lb3p):
    return pl.pallas_call(
        _final_body,
        out_shape=jax.ShapeDtypeStruct((NG, NF), F32),
    )(x1, x2, x3, lW1, lb1.reshape(1, NF), lW2p, lb2p, lW3p, lb3p)


def _layer(h, es, ed, W, b, pvec, nr):
    degp = _make_deg(nr)(ed)
    hh = _tc_mm(h, W, b)
    n = h.shape[0]
    d0 = lax.slice(degp, (0, 0), (n, 1))
    d1 = lax.slice(degp, (nr, 0), (nr + n, 1))
    hs, rn, ideg = _tc_scale(hh, d0, d1)
    aggp = _make_agg(nr)(es, ed, hs)
    score, hr = _tc_post(aggp[:n], aggp[nr:nr + n], hh, rn, ideg, pvec)
    return score, hr


def _pool(score, hr, batch_old, es, ed, n_old, nrm, k):
    kpad = _rup(k, 256)
    pad = kpad - k
    vals, perm = lax.top_k(score[:n_old, 0], k)
    spread = jnp.arange(pad, dtype=I32) % NDUM
    perm_s = jnp.concatenate([perm, n_old + spread])
    perm_g = jnp.concatenate([perm, spread])
    posk = jnp.concatenate([jnp.arange(k, dtype=I32),
                            jnp.full((pad,), -1, I32)])
    nb = batch_old.shape[0]
    xp, b2, esn, edn = _make_pool(nrm, kpad, nb, k)(
        perm_s, perm_g, posk, hr, batch_old, es, ed)
    vals_col = jnp.concatenate([vals, jnp.zeros((pad,), F32)]).reshape(kpad, 1)
    return xp, vals_col, b2, esn, edn


def kernel(x, edge_index, batch, W1, b1, p1, W2, b2, p2, W3, b3, p3,
           lW1, lb1, lW2, lb2, lW3, lb3):
    src = edge_index[0].astype(I32)
    dst = edge_index[1].astype(I32)
    batch = batch.astype(I32)

    nr1 = _rup(N0 + NDUM, 256)
    nr2 = _rup(max(K1 + NDUM, _rup(K1, 256)), 256)
    nr3 = _rup(max(K2 + NDUM, _rup(K2, 256)), 256)

    score1, hr1 = _layer(x, src, dst, W1, b1, p1, nr1)
    xp1, v1c, b21, es2, ed2 = _pool(score1, hr1, batch, src, dst,
                                    N0, nr1, K1)
    xn1, x1 = _tc_read(xp1, v1c, b21, K1)

    score2, hr2 = _layer(xn1, es2, ed2, W2, b2, p2, nr2)
    xp2, v2c, b22, es3, ed3 = _pool(score2, hr2, b21, es2, ed2,
                                    K1, nr2, K2)
    xn2, x2 = _tc_read(xp2, v2c, b22, K2)

    _, hr3 = _layer(xn2, es3, ed3, W3, b3, p3, nr3)
    big = jnp.full((xn2.shape[0], 1), 20.0, F32)
    _, x3 = _tc_read(hr3, big, b22, K2)

    lW2p = jnp.pad(lW2, ((0, 0), (0, NF - lW2.shape[1])))
    lb2p = jnp.pad(lb2, (0, NF - lb2.shape[0])).reshape(1, NF)
    lW3p = jnp.pad(lW3, ((0, NF - lW3.shape[0]), (0, NF - lW3.shape[1])))
    lb3p = jnp.concatenate(
        [lb3, jnp.full((NF - lb3.shape[0],), -1e30, F32)]).reshape(1, NF)
    out = _tc_final(x1, x2, x3, lW1, lb1, lW2p, lb2p, lW3p, lb3p)
    return out[:, :lb3.shape[0]]
